# Initial kernel scaffold; baseline (speedup 1.0000x reference)
import jax
import jax.numpy as jnp
from jax.experimental import pallas as pl


def kernel(state, action, edge_index, edge_attr, W_in, b_in, W_taps1, b1, W_taps2, b2, W_out, b_out):
    raise NotImplementedError("write your pallas kernel here")


# speedup vs baseline: 11.6078x; 11.6078x over previous
import functools

import jax
import jax.numpy as jnp
from jax import lax
from jax.experimental import pallas as pl
from jax.experimental.pallas import tpu as pltpu
from jax.experimental.pallas import tpu_sc as plsc

N = 100000
E = 1600000
DS = 96
DA = 32
C = 32
H = 16
K = 4

NSUB = 16
NCORE = 2

LANE = 128
NR = 12544
E_PAD = NR * LANE
CR = 8
ROWS_W = NR // NSUB
NCHUNK = ROWS_W // CR
ROWS_D = NR // (NSUB * NCORE)
NCHUNK_D = ROWS_D // CR

N_PAD = 100352
NODES_W = N_PAD // NSUB
CPR = 128
NCOPY = NODES_W // CPR

_mesh = plsc.VectorSubcoreMesh(core_axis_name="c", subcore_axis_name="s")
_sc_params = pltpu.CompilerParams(use_tc_tiling_on_sc=False)


def _zero_acc(dbuf, acc, s):
    def zb(i, carry):
        dbuf[i] = jnp.zeros((H,), jnp.float32)
        return carry
    lax.fori_loop(0, CPR, zb, 0, unroll=8)

    def qb(q, carry):
        rb = pl.multiple_of(s * NODES_W + q * CPR, CPR)
        pltpu.sync_copy(dbuf, acc.at[pl.ds(rb, CPR)])
        return carry
    lax.fori_loop(0, NCOPY, qb, 0)


def _scale_rows(rows, eav, nrows):
    def gb(g, carry):
        base = pl.multiple_of(g * H, H)
        ev = eav[pl.ds(base, H)]
        for j in range(H):
            i = base + j
            rows[i] = rows[i] * jnp.full((H,), ev[j], jnp.float32)
        return carry
    lax.fori_loop(0, nrows // H, gb, 0)


def _copy_out(acc, sbuf, dbuf, d2buf, dinv_hbm, dinv2_hbm, z_hbm, un_hbm, s):
    def qbody(q, carry):
        rb = pl.multiple_of(s * NODES_W + q * CPR, CPR)
        pltpu.sync_copy(acc.at[pl.ds(rb, CPR)], sbuf)
        pltpu.sync_copy(dinv_hbm.at[pl.ds(rb, CPR)], dbuf)
        pltpu.sync_copy(dinv2_hbm.at[pl.ds(rb, CPR)], d2buf)

        def rbody(i, carry2):
            sv = sbuf[i]
            dbuf[i] = sv * dbuf[i]
            d2buf[i] = sv * d2buf[i]
            return carry2
        lax.fori_loop(0, CPR, rbody, 0, unroll=8)
        pltpu.sync_copy(dbuf, z_hbm.at[pl.ds(rb, CPR)])
        pltpu.sync_copy(d2buf, un_hbm.at[pl.ds(rb, CPR)])
        return carry
    lax.fori_loop(0, NCOPY, qbody, 0)


@functools.partial(
    pl.kernel,
    out_type=(
        jax.ShapeDtypeStruct((N_PAD, H), jnp.float32),
        jax.ShapeDtypeStruct((N_PAD, H), jnp.float32),
        jax.ShapeDtypeStruct((N_PAD, H), jnp.float32),
        jax.ShapeDtypeStruct((N_PAD, H), jnp.float32),
    ),
    mesh=_mesh,
    scratch_types=(
        pltpu.VMEM((CR, LANE), jnp.int32),
        pltpu.VMEM((CR, LANE), jnp.int32),
        pltpu.VMEM((CR * LANE,), jnp.float32),
        pltpu.VMEM((CR * LANE, H), jnp.float32),
        pltpu.VMEM_SHARED((N_PAD, H), jnp.float32),
        pltpu.VMEM((CPR, H), jnp.float32),
        pltpu.VMEM((CPR, H), jnp.float32),
        pltpu.VMEM((CPR, H), jnp.float32),
        pltpu.SemaphoreType.DMA,
    ),
    compiler_params=_sc_params,
)
def _sc_step(ua_hbm, ub_hbm, src_hbm, dst_hbm, ea_hbm, dinv_hbm, dinv2_hbm,
             za_hbm, zb_hbm, una_hbm, unb_hbm,
             srcv, dstv, eav, rows, acc, sbuf, dbuf, d2buf, sem):
    c = lax.axis_index("c")
    s = lax.axis_index("s")

    def run(u_hbm, z_hbm, un_hbm):
        _zero_acc(dbuf, acc, s)
        plsc.subcore_barrier()
        row0 = s * ROWS_W

        def chunk(t, carry):
            a = row0 + t * CR
            pltpu.sync_copy(src_hbm.at[pl.ds(a, CR)], srcv)
            pltpu.sync_copy(dst_hbm.at[pl.ds(a, CR)], dstv)
            pltpu.sync_copy(ea_hbm.at[pl.ds(a * LANE, CR * LANE)], eav)
            descs = []
            for r in range(CR):
                descs.append(pltpu.async_copy(
                    u_hbm.at[srcv.at[r]],
                    rows.at[pl.ds(r * LANE, LANE)], sem))
            for d in descs:
                d.wait()
            _scale_rows(rows, eav, CR * LANE)
            for r in range(CR):
                pltpu.sync_copy(rows.at[pl.ds(r * LANE, LANE)],
                                acc.at[dstv.at[r]], add=True)
            return carry
        lax.fori_loop(0, NCHUNK, chunk, 0)
        plsc.subcore_barrier()
        _copy_out(acc, sbuf, dbuf, d2buf, dinv_hbm, dinv2_hbm, z_hbm, un_hbm, s)

    @pl.when(c == 0)
    def _():
        run(ua_hbm, za_hbm, una_hbm)

    @pl.when(c == 1)
    def _():
        run(ub_hbm, zb_hbm, unb_hbm)


@functools.partial(
    pl.kernel,
    out_type=(
        jax.ShapeDtypeStruct((N_PAD, H), jnp.float32),
        jax.ShapeDtypeStruct((N_PAD, H), jnp.float32),
    ),
    mesh=_mesh,
    scratch_types=(
        pltpu.VMEM((CR, LANE), jnp.int32),
        pltpu.VMEM((CR * LANE,), jnp.float32),
        pltpu.VMEM((CR * LANE, H), jnp.float32),
        pltpu.VMEM_SHARED((N_PAD, H), jnp.float32),
        pltpu.VMEM((CPR, H), jnp.float32),
        pltpu.SemaphoreType.DMA,
    ),
    compiler_params=_sc_params,
)
def _sc_deg(dst_hbm, ea_hbm, dega_hbm, degb_hbm,
            dstv, eav, rows, acc, sbuf, sem):
    c = lax.axis_index("c")
    s = lax.axis_index("s")
    wid = s * NCORE + c

    _zero_acc(sbuf, acc, s)
    plsc.subcore_barrier()
    row0 = wid * ROWS_D

    def chunk(t, carry):
        a = row0 + t * CR
        pltpu.sync_copy(dst_hbm.at[pl.ds(a, CR)], dstv)
        pltpu.sync_copy(ea_hbm.at[pl.ds(a * LANE, CR * LANE)], eav)

        def eb(g, cc):
            base = pl.multiple_of(g * H, H)
            ev = eav[pl.ds(base, H)]
            for j in range(H):
                rows[base + j] = jnp.full((H,), ev[j], jnp.float32)
            return cc
        lax.fori_loop(0, CR * LANE // H, eb, 0)
        for r in range(CR):
            pltpu.sync_copy(rows.at[pl.ds(r * LANE, LANE)],
                            acc.at[dstv.at[r]], add=True)
        return carry
    lax.fori_loop(0, NCHUNK_D, chunk, 0)
    plsc.subcore_barrier()

    def out(deg_hbm):
        def qb(q, carry):
            rb = pl.multiple_of(s * NODES_W + q * CPR, CPR)
            pltpu.sync_copy(acc.at[pl.ds(rb, CPR)], sbuf)
            pltpu.sync_copy(sbuf, deg_hbm.at[pl.ds(rb, CPR)])
            return carry
        lax.fori_loop(0, NCOPY, qb, 0)

    @pl.when(c == 0)
    def _():
        out(dega_hbm)

    @pl.when(c == 1)
    def _():
        out(degb_hbm)


BLK = 1000
NBLK = N // BLK


def _leaky(x):
    return jnp.where(x >= 0, x, 0.01 * x)


def _prep_body(state_ref, action_ref, dega_ref, degb_ref, win_ref, bin_ref,
               h_ref, ua_ref, ub_ref, dinv_ref, dinv2_ref):
    deg = dega_ref[:, :1] + degb_ref[:, :1]
    dinv = jnp.where(deg > 0, lax.rsqrt(jnp.maximum(deg, 1e-30)), 0.0)
    h = _leaky(state_ref[...] @ win_ref[:DS] + action_ref[...] @ win_ref[DS:]
               + bin_ref[...])
    h_ref[...] = h
    u = dinv * h
    ua_ref[...] = u[:, :H]
    ub_ref[...] = u[:, H:]
    dinv_ref[...] = jnp.broadcast_to(dinv, (BLK, H))
    dinv2_ref[...] = jnp.broadcast_to(dinv * dinv, (BLK, H))


def _tc_prep(state, action, dega, degb, W_in, b_in):
    row = lambda i: (i, 0)
    return pl.pallas_call(
        _prep_body,
        grid=(NBLK,),
        in_specs=[
            pl.BlockSpec((BLK, DS), row),
            pl.BlockSpec((BLK, DA), row),
            pl.BlockSpec((BLK, H), row),
            pl.BlockSpec((BLK, H), row),
            pl.BlockSpec((DS + DA, C), lambda i: (0, 0)),
            pl.BlockSpec((1, C), lambda i: (0, 0)),
        ],
        out_specs=[
            pl.BlockSpec((BLK, C), row),
            pl.BlockSpec((BLK, H), row),
            pl.BlockSpec((BLK, H), row),
            pl.BlockSpec((BLK, H), row),
            pl.BlockSpec((BLK, H), row),
        ],
        out_shape=[
            jax.ShapeDtypeStruct((N, C), jnp.float32),
            jax.ShapeDtypeStruct((N_PAD, H), jnp.float32),
            jax.ShapeDtypeStruct((N_PAD, H), jnp.float32),
            jax.ShapeDtypeStruct((N_PAD, H), jnp.float32),
            jax.ShapeDtypeStruct((N_PAD, H), jnp.float32),
        ],
    )(state, action, dega, degb, W_in, b_in)


def _layer_body(h_ref, z_refs, wt_ref, b_ref, dinv_ref,
                hn_ref, ua_ref, ub_ref):
    out = h_ref[...] @ wt_ref[0]
    for k in range(K):
        zk = jnp.concatenate([z_refs[2 * k][...], z_refs[2 * k + 1][...]],
                             axis=1)
        out = out + zk @ wt_ref[k + 1]
    hn = _leaky(out + b_ref[...])
    hn_ref[...] = hn
    dinv = dinv_ref[...]
    ua_ref[...] = dinv * hn[:, :H]
    ub_ref[...] = dinv * hn[:, H:]


def _tc_layer(h, zs, W_taps, b, dinv):
    row = lambda i: (i, 0)

    def body(h_ref, z0, z1, z2, z3, z4, z5, z6, z7, wt_ref, b_ref, dinv_ref,
             hn_ref, ua_ref, ub_ref):
        _layer_body(h_ref, (z0, z1, z2, z3, z4, z5, z6, z7), wt_ref, b_ref,
                    dinv_ref, hn_ref, ua_ref, ub_ref)

    return pl.pallas_call(
        body,
        grid=(NBLK,),
        in_specs=[pl.BlockSpec((BLK, C), row)]
        + [pl.BlockSpec((BLK, H), row)] * (2 * K)
        + [pl.BlockSpec((K + 1, C, C), lambda i: (0, 0, 0)),
           pl.BlockSpec((1, C), lambda i: (0, 0)),
           pl.BlockSpec((BLK, H), row)],
        out_specs=[
            pl.BlockSpec((BLK, C), row),
            pl.BlockSpec((BLK, H), row),
            pl.BlockSpec((BLK, H), row),
        ],
        out_shape=[
            jax.ShapeDtypeStruct((N, C), jnp.float32),
            jax.ShapeDtypeStruct((N_PAD, H), jnp.float32),
            jax.ShapeDtypeStruct((N_PAD, H), jnp.float32),
        ],
    )(h, *zs, W_taps, b, dinv)


def _final_body(h_ref, z0, z1, z2, z3, z4, z5, z6, z7, wt_ref, b_ref,
                wo_ref, bo_ref, y_ref):
    z_refs = (z0, z1, z2, z3, z4, z5, z6, z7)
    out = h_ref[...] @ wt_ref[0]
    for k in range(K):
        zk = jnp.concatenate([z_refs[2 * k][...], z_refs[2 * k + 1][...]],
                             axis=1)
        out = out + zk @ wt_ref[k + 1]
    h2 = _leaky(out + b_ref[...])
    y_ref[...] = h2 @ wo_ref[...] + bo_ref[...]


def _tc_final(h, zs, W_taps, b, W_out, b_out):
    row = lambda i: (i, 0)
    return pl.pallas_call(
        _final_body,
        grid=(NBLK,),
        in_specs=[pl.BlockSpec((BLK, C), row)]
        + [pl.BlockSpec((BLK, H), row)] * (2 * K)
        + [pl.BlockSpec((K + 1, C, C), lambda i: (0, 0, 0)),
           pl.BlockSpec((1, C), lambda i: (0, 0)),
           pl.BlockSpec((C, 1), lambda i: (0, 0)),
           pl.BlockSpec((1, 1), lambda i: (0, 0))],
        out_specs=pl.BlockSpec((BLK, 1), row),
        out_shape=jax.ShapeDtypeStruct((N, 1), jnp.float32),
    )(h, *zs, W_taps, b, W_out, b_out)


def kernel(state, action, edge_index, edge_attr, W_in, b_in,
           W_taps1, b1, W_taps2, b2, W_out, b_out):
    pad = E_PAD - E
    src = jnp.concatenate([edge_index[0], jnp.zeros((pad,), jnp.int32)])
    dst = jnp.concatenate([edge_index[1], jnp.zeros((pad,), jnp.int32)])
    ea = jnp.concatenate([edge_attr, jnp.zeros((pad,), jnp.float32)])
    src2 = src.reshape(NR, LANE)
    dst2 = dst.reshape(NR, LANE)

    dega, degb = _sc_deg(dst2, ea)
    h0, ua, ub, dinv, dinv2 = _tc_prep(state, action, dega, degb, W_in,
                                       b_in.reshape(1, C))

    zs = []
    for _ in range(K):
        za, zb, ua, ub = _sc_step(ua, ub, src2, dst2, ea, dinv, dinv2)
        zs += [za, zb]
    h1, ua, ub = _tc_layer(h0, zs, W_taps1, b1.reshape(1, C), dinv)

    zs = []
    for _ in range(K):
        za, zb, ua, ub = _sc_step(ua, ub, src2, dst2, ea, dinv, dinv2)
        zs += [za, zb]
    y = _tc_final(h1, zs, W_taps2, b2.reshape(1, C), W_out, b_out.reshape(1, 1))
    return y.reshape(N)


# speedup vs baseline: 16.5679x; 1.4273x over previous
import functools

import jax
import jax.numpy as jnp
from jax import lax
from jax.experimental import pallas as pl
from jax.experimental.pallas import tpu as pltpu
from jax.experimental.pallas import tpu_sc as plsc

N = 100000
E = 1600000
DS = 96
DA = 32
C = 32
H = 16
K = 4

NSUB = 16
NCORE = 2

LANE = 128
NR = 12544
E_PAD = NR * LANE
CR = 4
NB = NR // CR
ROWS_W = NR // NSUB
NCHUNK = ROWS_W // CR
ROWS_D = NR // (NSUB * NCORE)
NCHUNK_D = ROWS_D // CR

N_PAD = 100352
NODES_W = N_PAD // NSUB
CPR = 128
NCOPY = NODES_W // CPR

_mesh = plsc.VectorSubcoreMesh(core_axis_name="c", subcore_axis_name="s")
_sc_params = pltpu.CompilerParams(use_tc_tiling_on_sc=False)


def _zero_acc(dbuf, acc, s):
    def zb(i, carry):
        dbuf[i] = jnp.zeros((H,), jnp.float32)
        return carry
    lax.fori_loop(0, CPR, zb, 0, unroll=8)

    def qb(q, carry):
        rb = pl.multiple_of(s * NODES_W + q * CPR, CPR)
        pltpu.sync_copy(dbuf, acc.at[pl.ds(rb, CPR)])
        return carry
    lax.fori_loop(0, NCOPY, qb, 0)


def _copy_out(acc, sbuf, dbuf, d2buf, dinv_hbm, dinv2_hbm, z_hbm, un_hbm, s):
    def qbody(q, carry):
        rb = pl.multiple_of(s * NODES_W + q * CPR, CPR)
        pltpu.sync_copy(acc.at[pl.ds(rb, CPR)], sbuf)
        pltpu.sync_copy(dinv_hbm.at[pl.ds(rb, CPR)], dbuf)
        pltpu.sync_copy(dinv2_hbm.at[pl.ds(rb, CPR)], d2buf)

        def rbody(i, carry2):
            sv = sbuf[i]
            dbuf[i] = sv * dbuf[i]
            d2buf[i] = sv * d2buf[i]
            return carry2
        lax.fori_loop(0, CPR, rbody, 0, unroll=8)
        pltpu.sync_copy(dbuf, z_hbm.at[pl.ds(rb, CPR)])
        pltpu.sync_copy(d2buf, un_hbm.at[pl.ds(rb, CPR)])
        return carry
    lax.fori_loop(0, NCOPY, qbody, 0)


@functools.partial(
    pl.kernel,
    out_type=tuple(
        [jax.ShapeDtypeStruct((N_PAD, H), jnp.float32)] * (2 * K)
        + [jax.ShapeDtypeStruct((N_PAD, H), jnp.float32)] * 2
    ),
    mesh=_mesh,
    scratch_types=(
        [pltpu.VMEM((2, CR, LANE), jnp.int32)] * 2
        + [pltpu.VMEM((CR, LANE), jnp.int32)] * 2
        + [pltpu.VMEM((CR * LANE,), jnp.float32)] * 2
        + [pltpu.VMEM((CR, LANE, H), jnp.float32)] * 2
        + [
            pltpu.VMEM_SHARED((N_PAD, H), jnp.float32),
            pltpu.VMEM((CPR, H), jnp.float32),
            pltpu.VMEM((CPR, H), jnp.float32),
            pltpu.VMEM((CPR, H), jnp.float32),
        ]
        + [pltpu.SemaphoreType.DMA] * 6
    ),
    compiler_params=_sc_params,
)
def _sc_layer(ua_hbm, ub_hbm, sd_hbm, ea_hbm, dinv_hbm, dinv2_hbm,
              z1a, z1b, z2a, z2b, z3a, z3b, z4a, z4b, una_hbm, unb_hbm,
              sdv0, sdv1, sidx0, sidx1, eav0, eav1, rows0, rows1,
              acc, sbuf, dbuf, d2buf,
              semIn0, semIn1, semG0, semG1, semS0, semS1):
    c = lax.axis_index("c")
    s = lax.axis_index("s")
    slots = (
        (sdv0, sidx0, eav0, rows0, semIn0, semG0, semS0),
        (sdv1, sidx1, eav1, rows1, semIn1, semG1, semS1),
    )

    def one_step(u_pair, z_pair, un_pair):
        _zero_acc(dbuf, acc, s)
        plsc.subcore_barrier()
        blk0 = s * NCHUNK

        def issue_in(i, X):
            sdv, _, eav, _, semIn, _, _ = X
            pltpu.async_copy(sd_hbm.at[blk0 + i], sdv, semIn)
            a = (blk0 + i) * CR * LANE
            pltpu.async_copy(ea_hbm.at[pl.ds(a, CR * LANE)], eav, semIn)

        def wait_in(i, X):
            sdv, _, eav, _, semIn, _, _ = X
            pltpu.make_async_copy(sd_hbm.at[blk0 + i], sdv, semIn).wait()
            a = (blk0 + i) * CR * LANE
            pltpu.make_async_copy(ea_hbm.at[pl.ds(a, CR * LANE)], eav,
                                  semIn).wait()

        def fire_gather(X):
            sdv, _, _, rows, _, semG, _ = X

            @pl.when(c == 0)
            def _():
                for r in range(CR):
                    pltpu.async_copy(u_pair[0].at[sdv.at[0, r]],
                                     rows.at[r], semG)

            @pl.when(c == 1)
            def _():
                for r in range(CR):
                    pltpu.async_copy(u_pair[1].at[sdv.at[0, r]],
                                     rows.at[r], semG)

        def wait_gather(X):
            sdv, _, _, rows, _, semG, _ = X
            for r in range(CR):
                pltpu.make_async_copy(u_pair[0].at[sdv.at[0, r]],
                                      rows.at[r], semG).wait()

        def fire_scatter(X):
            sdv, sidx, _, rows, _, _, semS = X
            for r in range(CR):
                def cp(k, carry):
                    o = pl.multiple_of(k * H, H)
                    sidx[r, pl.ds(o, H)] = sdv[1, r, pl.ds(o, H)]
                    return carry
                lax.fori_loop(0, LANE // H, cp, 0, unroll=8)
            for r in range(CR):
                pltpu.async_copy(rows.at[r], acc.at[sidx.at[r]], semS,
                                 add=True)

        def wait_scatter(X):
            _, sidx, _, rows, _, _, semS = X
            for r in range(CR):
                pltpu.make_async_copy(rows.at[r], acc.at[sidx.at[r]],
                                      semS).wait()

        def scale_rows(X):
            _, _, eav, rows, _, _, _ = X
            for r in range(CR):
                def gb(g, carry):
                    base = pl.multiple_of(g * H, H)
                    ev = eav[pl.ds(r * LANE + base, H)]
                    for j in range(H):
                        rows[r, base + j] = (
                            rows[r, base + j]
                            * jnp.full((H,), ev[j], jnp.float32))
                    return carry
                lax.fori_loop(0, LANE // H, gb, 0)

        def turn(i, xp):
            X, Y = slots[xp], slots[1 - xp]

            @pl.when(i <= NCHUNK - 1)
            def _():
                @pl.when(i >= 2)
                def _():
                    wait_scatter(X)
                wait_in(i, X)
                fire_gather(X)

            wait_gather(Y)
            scale_rows(Y)
            fire_scatter(Y)

            @pl.when(i <= NCHUNK - 2)
            def _():
                issue_in(i + 1, Y)

        issue_in(0, slots[0])
        wait_in(0, slots[0])
        fire_gather(slots[0])
        issue_in(1, slots[1])

        def pair(q, carry):
            turn(1 + 2 * q, 1)
            turn(2 + 2 * q, 0)
            return carry
        lax.fori_loop(0, NCHUNK // 2, pair, 0)

        wait_scatter(slots[0])
        wait_scatter(slots[1])

        plsc.subcore_barrier()

        def co(half_refs):
            _copy_out(acc, sbuf, dbuf, d2buf, dinv_hbm, dinv2_hbm,
                      half_refs[0], half_refs[1], s)

        @pl.when(c == 0)
        def _():
            co((z_pair[0], un_pair[0]))

        @pl.when(c == 1)
        def _():
            co((z_pair[1], un_pair[1]))

    zs = ((z1a, z1b), (z2a, z2b), (z3a, z3b), (z4a, z4b))
    un = (una_hbm, unb_hbm)
    one_step((ua_hbm, ub_hbm), zs[0], un)
    for k in range(1, K):
        one_step(un, zs[k], un)


@functools.partial(
    pl.kernel,
    out_type=(
        jax.ShapeDtypeStruct((N_PAD, H), jnp.float32),
        jax.ShapeDtypeStruct((N_PAD, H), jnp.float32),
    ),
    mesh=_mesh,
    scratch_types=(
        pltpu.VMEM((CR, LANE), jnp.int32),
        pltpu.VMEM((CR * LANE,), jnp.float32),
        pltpu.VMEM((CR, LANE, H), jnp.float32),
        pltpu.VMEM_SHARED((N_PAD, H), jnp.float32),
        pltpu.VMEM((CPR, H), jnp.float32),
        pltpu.SemaphoreType.DMA,
    ),
    compiler_params=_sc_params,
)
def _sc_deg(sd_hbm, ea_hbm, dega_hbm, degb_hbm,
            dstv, eav, rows, acc, sbuf, sem):
    c = lax.axis_index("c")
    s = lax.axis_index("s")
    wid = s * NCORE + c

    _zero_acc(sbuf, acc, s)
    plsc.subcore_barrier()
    blk0 = wid * NCHUNK_D

    def chunk(t, carry):
        b = blk0 + t
        pltpu.sync_copy(sd_hbm.at[b, 1], dstv)
        a = b * CR * LANE
        pltpu.sync_copy(ea_hbm.at[pl.ds(a, CR * LANE)], eav)
        for r in range(CR):
            def gb(g, cc):
                base = pl.multiple_of(g * H, H)
                ev = eav[pl.ds(r * LANE + base, H)]
                for j in range(H):
                    rows[r, base + j] = jnp.full((H,), ev[j], jnp.float32)
                return cc
            lax.fori_loop(0, LANE // H, gb, 0)
        for r in range(CR):
            pltpu.sync_copy(rows.at[r], acc.at[dstv.at[r]], add=True)
        return carry
    lax.fori_loop(0, NCHUNK_D, chunk, 0)
    plsc.subcore_barrier()

    def out(deg_hbm):
        def qb(q, carry):
            rb = pl.multiple_of(s * NODES_W + q * CPR, CPR)
            pltpu.sync_copy(acc.at[pl.ds(rb, CPR)], sbuf)
            pltpu.sync_copy(sbuf, deg_hbm.at[pl.ds(rb, CPR)])
            return carry
        lax.fori_loop(0, NCOPY, qb, 0)

    @pl.when(c == 0)
    def _():
        out(dega_hbm)

    @pl.when(c == 1)
    def _():
        out(degb_hbm)


BLK = 1000
NBLK = N // BLK


def _leaky(x):
    return jnp.where(x >= 0, x, 0.01 * x)


def _prep_body(state_ref, action_ref, dega_ref, degb_ref, win_ref, bin_ref,
               h_ref, ua_ref, ub_ref, dinv_ref, dinv2_ref):
    deg = dega_ref[:, :1] + degb_ref[:, :1]
    dinv = jnp.where(deg > 0, lax.rsqrt(jnp.maximum(deg, 1e-30)), 0.0)
    h = _leaky(state_ref[...] @ win_ref[:DS] + action_ref[...] @ win_ref[DS:]
               + bin_ref[...])
    h_ref[...] = h
    u = dinv * h
    ua_ref[...] = u[:, :H]
    ub_ref[...] = u[:, H:]
    dinv_ref[...] = jnp.broadcast_to(dinv, (BLK, H))
    dinv2_ref[...] = jnp.broadcast_to(dinv * dinv, (BLK, H))


def _tc_prep(state, action, dega, degb, W_in, b_in):
    row = lambda i: (i, 0)
    return pl.pallas_call(
        _prep_body,
        grid=(NBLK,),
        in_specs=[
            pl.BlockSpec((BLK, DS), row),
            pl.BlockSpec((BLK, DA), row),
            pl.BlockSpec((BLK, H), row),
            pl.BlockSpec((BLK, H), row),
            pl.BlockSpec((DS + DA, C), lambda i: (0, 0)),
            pl.BlockSpec((1, C), lambda i: (0, 0)),
        ],
        out_specs=[
            pl.BlockSpec((BLK, C), row),
            pl.BlockSpec((BLK, H), row),
            pl.BlockSpec((BLK, H), row),
            pl.BlockSpec((BLK, H), row),
            pl.BlockSpec((BLK, H), row),
        ],
        out_shape=[
            jax.ShapeDtypeStruct((N, C), jnp.float32),
            jax.ShapeDtypeStruct((N_PAD, H), jnp.float32),
            jax.ShapeDtypeStruct((N_PAD, H), jnp.float32),
            jax.ShapeDtypeStruct((N_PAD, H), jnp.float32),
            jax.ShapeDtypeStruct((N_PAD, H), jnp.float32),
        ],
    )(state, action, dega, degb, W_in, b_in)


def _layer_body(h_ref, z_refs, wt_ref, b_ref, dinv_ref,
                hn_ref, ua_ref, ub_ref):
    out = h_ref[...] @ wt_ref[0]
    for k in range(K):
        zk = jnp.concatenate([z_refs[2 * k][...], z_refs[2 * k + 1][...]],
                             axis=1)
        out = out + zk @ wt_ref[k + 1]
    hn = _leaky(out + b_ref[...])
    hn_ref[...] = hn
    dinv = dinv_ref[...]
    ua_ref[...] = dinv * hn[:, :H]
    ub_ref[...] = dinv * hn[:, H:]


def _tc_layer(h, zs, W_taps, b, dinv):
    row = lambda i: (i, 0)

    def body(h_ref, z0, z1, z2, z3, z4, z5, z6, z7, wt_ref, b_ref, dinv_ref,
             hn_ref, ua_ref, ub_ref):
        _layer_body(h_ref, (z0, z1, z2, z3, z4, z5, z6, z7), wt_ref, b_ref,
                    dinv_ref, hn_ref, ua_ref, ub_ref)

    return pl.pallas_call(
        body,
        grid=(NBLK,),
        in_specs=[pl.BlockSpec((BLK, C), row)]
        + [pl.BlockSpec((BLK, H), row)] * (2 * K)
        + [pl.BlockSpec((K + 1, C, C), lambda i: (0, 0, 0)),
           pl.BlockSpec((1, C), lambda i: (0, 0)),
           pl.BlockSpec((BLK, H), row)],
        out_specs=[
            pl.BlockSpec((BLK, C), row),
            pl.BlockSpec((BLK, H), row),
            pl.BlockSpec((BLK, H), row),
        ],
        out_shape=[
            jax.ShapeDtypeStruct((N, C), jnp.float32),
            jax.ShapeDtypeStruct((N_PAD, H), jnp.float32),
            jax.ShapeDtypeStruct((N_PAD, H), jnp.float32),
        ],
    )(h, *zs, W_taps, b, dinv)


def _final_body(h_ref, z0, z1, z2, z3, z4, z5, z6, z7, wt_ref, b_ref,
                wo_ref, bo_ref, y_ref):
    z_refs = (z0, z1, z2, z3, z4, z5, z6, z7)
    out = h_ref[...] @ wt_ref[0]
    for k in range(K):
        zk = jnp.concatenate([z_refs[2 * k][...], z_refs[2 * k + 1][...]],
                             axis=1)
        out = out + zk @ wt_ref[k + 1]
    h2 = _leaky(out + b_ref[...])
    y_ref[...] = h2 @ wo_ref[...] + bo_ref[...]


def _tc_final(h, zs, W_taps, b, W_out, b_out):
    row = lambda i: (i, 0)
    return pl.pallas_call(
        _final_body,
        grid=(NBLK,),
        in_specs=[pl.BlockSpec((BLK, C), row)]
        + [pl.BlockSpec((BLK, H), row)] * (2 * K)
        + [pl.BlockSpec((K + 1, C, C), lambda i: (0, 0, 0)),
           pl.BlockSpec((1, C), lambda i: (0, 0)),
           pl.BlockSpec((C, 1), lambda i: (0, 0)),
           pl.BlockSpec((1, 1), lambda i: (0, 0))],
        out_specs=pl.BlockSpec((BLK, 1), row),
        out_shape=jax.ShapeDtypeStruct((N, 1), jnp.float32),
    )(h, *zs, W_taps, b, W_out, b_out)


def kernel(state, action, edge_index, edge_attr, W_in, b_in,
           W_taps1, b1, W_taps2, b2, W_out, b_out):
    pad = E_PAD - E
    src = jnp.concatenate([edge_index[0], jnp.zeros((pad,), jnp.int32)])
    dst = jnp.concatenate([edge_index[1], jnp.zeros((pad,), jnp.int32)])
    ea = jnp.concatenate([edge_attr, jnp.zeros((pad,), jnp.float32)])
    sd = jnp.stack([src.reshape(NB, CR, LANE), dst.reshape(NB, CR, LANE)],
                   axis=1)

    dega, degb = _sc_deg(sd, ea)
    h0, ua, ub, dinv, dinv2 = _tc_prep(state, action, dega, degb, W_in,
                                       b_in.reshape(1, C))

    z1a, z1b, z2a, z2b, z3a, z3b, z4a, z4b, ua, ub = _sc_layer(
        ua, ub, sd, ea, dinv, dinv2)
    zs = [z1a, z1b, z2a, z2b, z3a, z3b, z4a, z4b]
    h1, ua, ub = _tc_layer(h0, zs, W_taps1, b1.reshape(1, C), dinv)

    z1a, z1b, z2a, z2b, z3a, z3b, z4a, z4b, ua, ub = _sc_layer(
        ua, ub, sd, ea, dinv, dinv2)
    zs = [z1a, z1b, z2a, z2b, z3a, z3b, z4a, z4b]
    y = _tc_final(h1, zs, W_taps2, b2.reshape(1, C), W_out, b_out.reshape(1, 1))
    return y.reshape(N)
